# Initial kernel scaffold; baseline (speedup 1.0000x reference)
import jax
import jax.numpy as jnp
from jax.experimental import pallas as pl


def kernel(x, edge_index, W0, b0, W1, b1, W2, b2, Dw0, Db0, Dw1, Db1, Dw2, Db2, mask_token):
    raise NotImplementedError("write your pallas kernel here")


# speedup vs baseline: 10.3779x; 10.3779x over previous
import functools

import jax
import jax.numpy as jnp
import numpy as np
from jax import lax
from jax.experimental import pallas as pl
from jax.experimental.pallas import tpu as pltpu
from jax.experimental.pallas import tpu_sc as plsc

N = 10000
E = 320000
D = 128
H = 128
NUM_MASK = 5000

N_PAD = 10240
NC = 2
NS = 16
NW = NC * NS
R_PT = N_PAD // NS
CH = 128
NCH = 79
E_PT = NCH * CH
E_PAD = NW * E_PT
G_PAD = 5120
G_PT = G_PAD // NW
GCH = 80

_EPAD_IDX = (np.arange(E_PAD - E, dtype=np.int32) % (N_PAD - N)) + N


def _threefry2x32_np(k1, k2, x0, x1):
    x0 = x0.astype(np.uint32).copy()
    x1 = x1.astype(np.uint32).copy()
    ks = [np.uint32(k1), np.uint32(k2), np.uint32(k1 ^ k2 ^ 0x1BD11BDA)]

    def rotl(v, r):
        return (v << r) | (v >> np.uint32(32 - r))

    rots = [(13, 15, 26, 6), (17, 29, 16, 24)]
    kidx = [(1, 2), (2, 0), (0, 1), (1, 2), (2, 0)]
    with np.errstate(over="ignore"):
        x0 = x0 + ks[0]
        x1 = x1 + ks[1]
        for i in range(5):
            for r in rots[i % 2]:
                x0 = x0 + x1
                x1 = rotl(x1, np.uint32(r))
                x1 = x0 ^ x1
            a, b = kidx[i]
            x0 = x0 + ks[a]
            x1 = x1 + ks[b] + np.uint32(i + 1)
    return x0, x1


def _permutation_np(seed, n):
    key = (np.uint32((seed >> 32) & 0xFFFFFFFF), np.uint32(seed & 0xFFFFFFFF))
    x = np.arange(n)
    num_rounds = int(np.ceil(3 * np.log(max(1, n))
                             / np.log(np.iinfo(np.uint32).max)))
    for _ in range(num_rounds):
        b1, b2 = _threefry2x32_np(key[0], key[1],
                                  np.array([0, 0]), np.array([0, 1]))
        key, sub = (b1[0], b2[0]), (b1[1], b2[1])
        h1, h2 = _threefry2x32_np(sub[0], sub[1],
                                  np.zeros(n, np.uint32),
                                  np.arange(n, dtype=np.uint32))
        x = x[np.argsort(h1 ^ h2, kind="stable")]
    return x


@functools.cache
def _mask_consts():
    perm = _permutation_np(42, N)
    mask_nodes = perm[:NUM_MASK].astype(np.int32)
    mask_vec = np.zeros((N_PAD, 1), np.float32)
    mask_vec[mask_nodes, 0] = 1.0
    gidx = np.concatenate(
        [mask_nodes, (np.arange(G_PAD - NUM_MASK, dtype=np.int32) * 83) % N]
    )
    return mask_vec, gidx

_f32 = jnp.float32


def _worker_id():
    return lax.axis_index("c") * NS + lax.axis_index("s")


@functools.cache
def _sc_kernels():
    mesh = plsc.VectorSubcoreMesh(core_axis_name="c", subcore_axis_name="s",
                                  num_cores=NC, num_subcores=NS)

    @functools.partial(
        pl.kernel,
        out_type=jax.ShapeDtypeStruct((NC, N_PAD, H), _f32),
        mesh=mesh,
        scratch_types=[
            pltpu.VMEM((CH,), jnp.int32),
            pltpu.VMEM((CH,), jnp.int32),
            pltpu.VMEM((CH, H), _f32),
            pltpu.VMEM_SHARED((N_PAD, H), _f32),
            pltpu.SemaphoreType.DMA,
        ],
    )
    def scatter_kernel(hs_hbm, src_hbm, dst_hbm, zeros_hbm, out_hbm,
                       sidx_v, didx_v, rows_v, acc_sh, sem):
        c = lax.axis_index("c")
        s = lax.axis_index("s")
        w = _worker_id()
        pltpu.sync_copy(zeros_hbm.at[pl.ds(s * R_PT, R_PT)],
                        acc_sh.at[pl.ds(s * R_PT, R_PT)])
        plsc.subcore_barrier()

        def body(j, carry):
            off = w * E_PT + j * CH
            pltpu.sync_copy(src_hbm.at[pl.ds(off, CH)], sidx_v)
            pltpu.sync_copy(dst_hbm.at[pl.ds(off, CH)], didx_v)
            pltpu.async_copy(hs_hbm.at[sidx_v], rows_v, sem).wait()
            pltpu.sync_copy(rows_v, acc_sh.at[didx_v], add=True)
            return carry

        lax.fori_loop(0, NCH, body, 0)
        plsc.subcore_barrier()
        pltpu.sync_copy(acc_sh.at[pl.ds(s * R_PT, R_PT)],
                        out_hbm.at[c, pl.ds(s * R_PT, R_PT)])

    @functools.partial(
        pl.kernel,
        out_type=(jax.ShapeDtypeStruct((G_PAD, D), _f32),
                  jax.ShapeDtypeStruct((G_PAD, D), _f32)),
        mesh=mesh,
        scratch_types=[
            pltpu.VMEM((GCH,), jnp.int32),
            pltpu.VMEM((GCH, D), _f32),
            pltpu.SemaphoreType.DMA,
        ],
    )
    def gather_kernel(x_hbm, r_hbm, gidx_hbm, out0_hbm, out1_hbm, idx_v, rows_v, sem):
        w = _worker_id()
        for part in range(G_PT // GCH):
            base = w * G_PT + part * GCH
            pltpu.sync_copy(gidx_hbm.at[pl.ds(base, GCH)], idx_v)
            pltpu.async_copy(x_hbm.at[idx_v], rows_v, sem).wait()
            pltpu.sync_copy(rows_v, out0_hbm.at[pl.ds(base, GCH)])
            pltpu.async_copy(r_hbm.at[idx_v], rows_v, sem).wait()
            pltpu.sync_copy(rows_v, out1_hbm.at[pl.ds(base, GCH)])

    return scatter_kernel, gather_kernel


R_TC = 256
G_TC = N_PAD // R_TC


def _row_spec(width):
    return pl.BlockSpec((R_TC, width), lambda i: (i, 0))


def _full_spec(shape):
    nd = len(shape)
    return pl.BlockSpec(shape, lambda i: (0,) * nd)


def _pre_body(x_ref, m_ref, mt_ref, w_ref, dp_ref, hs_ref, dinv_ref):
    i = pl.program_id(0)
    dp = dp_ref[...]
    deg = dp[0, :, 0:1] + dp[1, :, 0:1] + 1.0
    rows = lax.broadcasted_iota(jnp.int32, (R_TC, 1), 0) + i * R_TC
    dinv = lax.rsqrt(deg) * (rows < N).astype(_f32)
    m = m_ref[...]
    z0 = x_ref[...] * (1.0 - m) + mt_ref[...] * m
    h = jnp.dot(z0, w_ref[...], preferred_element_type=_f32)
    hs_ref[...] = h * dinv
    dinv_ref[...] = dinv


def _tc_pre(x_pad, mvec, mt, W0, deg_col):
    return pl.pallas_call(
        _pre_body,
        grid=(G_TC,),
        in_specs=[
            _row_spec(D),
            _row_spec(1),
            _full_spec((1, D)),
            _full_spec((D, H)),
            pl.BlockSpec((NC, R_TC, H), lambda i: (0, i, 0)),
        ],
        out_specs=(_row_spec(H), _row_spec(1)),
        out_shape=(jax.ShapeDtypeStruct((N_PAD, H), _f32),
                   jax.ShapeDtypeStruct((N_PAD, 1), _f32)),
    )(x_pad, mvec, mt, W0, deg_col)


def _mid_body(p_ref, hs_ref, dinv_ref, b_ref, w_ref, out_ref):
    p = p_ref[...]
    acc = p[0] + p[1] + hs_ref[...]
    dinv = dinv_ref[...]
    z = jnp.maximum(dinv * acc + b_ref[...], 0.0)
    out_ref[...] = jnp.dot(z, w_ref[...], preferred_element_type=_f32) * dinv


def _tc_mid(p, hs_prev, dinv, b_prev, W_next):
    return pl.pallas_call(
        _mid_body,
        grid=(G_TC,),
        in_specs=[
            pl.BlockSpec((NC, R_TC, H), lambda i: (0, i, 0)),
            _row_spec(H),
            _row_spec(1),
            _full_spec((1, H)),
            _full_spec((H, H)),
        ],
        out_specs=_row_spec(H),
        out_shape=jax.ShapeDtypeStruct((N_PAD, H), _f32),
    )(p, hs_prev, dinv, b_prev, W_next)


def _post_body(p_ref, hs_ref, dinv_ref, b_ref, w0_ref, c0_ref, w1_ref, c1_ref,
               w2_ref, c2_ref, out_ref):
    p = p_ref[...]
    acc = p[0] + p[1] + hs_ref[...]
    z = jnp.maximum(dinv_ref[...] * acc + b_ref[...], 0.0)
    r = jnp.maximum(jnp.dot(z, w0_ref[...], preferred_element_type=_f32)
                    + c0_ref[...], 0.0)
    r = jnp.maximum(jnp.dot(r, w1_ref[...], preferred_element_type=_f32)
                    + c1_ref[...], 0.0)
    out_ref[...] = (jnp.dot(r, w2_ref[...], preferred_element_type=_f32)
                    + c2_ref[...])


def _tc_post(p, hs3, dinv, b2, Dw0, Db0, Dw1, Db1, Dw2, Db2):
    return pl.pallas_call(
        _post_body,
        grid=(G_TC,),
        in_specs=[
            pl.BlockSpec((NC, R_TC, H), lambda i: (0, i, 0)),
            _row_spec(H),
            _row_spec(1),
            _full_spec((1, H)),
            _full_spec((H, H)),
            _full_spec((1, H)),
            _full_spec((H, H)),
            _full_spec((1, H)),
            _full_spec((H, D)),
            _full_spec((1, D)),
        ],
        out_specs=_row_spec(D),
        out_shape=jax.ShapeDtypeStruct((N_PAD, D), _f32),
    )(p, hs3, dinv, b2, Dw0, Db0, Dw1, Db1, Dw2, Db2)


def kernel(x, edge_index, W0, b0, W1, b1, W2, b2,
           Dw0, Db0, Dw1, Db1, Dw2, Db2, mask_token):
    x_pad = jnp.concatenate([x, jnp.zeros((N_PAD - N, D), _f32)], axis=0)
    src = jnp.concatenate([edge_index[0], jnp.asarray(_EPAD_IDX)])
    dst = jnp.concatenate([edge_index[1], jnp.asarray(_EPAD_IDX)])
    mask_vec, gidx_np = _mask_consts()
    mvec = jnp.asarray(mask_vec)
    gidx = jnp.asarray(gidx_np)
    onesH = jnp.ones((N_PAD, H), _f32)
    zerosH = jnp.zeros((N_PAD, H), _f32)

    _scatter_kernel, _gather_kernel = _sc_kernels()
    degp = _scatter_kernel(onesH, dst, dst, zerosH)
    hs1, dinv = _tc_pre(x_pad, mvec, mask_token, W0, degp)
    p1 = _scatter_kernel(hs1, src, dst, zerosH)
    hs2 = _tc_mid(p1, hs1, dinv, b0.reshape(1, H), W1)
    p2 = _scatter_kernel(hs2, src, dst, zerosH)
    hs3 = _tc_mid(p2, hs2, dinv, b1.reshape(1, H), W2)
    p3 = _scatter_kernel(hs3, src, dst, zerosH)
    r = _tc_post(p3, hs3, dinv, b2.reshape(1, H),
                 Dw0, Db0.reshape(1, H), Dw1, Db1.reshape(1, H),
                 Dw2, Db2.reshape(1, D))
    out0, out1 = _gather_kernel(x_pad, r, gidx)
    return (out0[:NUM_MASK], out1[:NUM_MASK])


# speedup vs baseline: 18.8751x; 1.8188x over previous
import functools

import jax
import jax.numpy as jnp
import numpy as np
from jax import lax
from jax.experimental import pallas as pl
from jax.experimental.pallas import tpu as pltpu
from jax.experimental.pallas import tpu_sc as plsc

N = 10000
E = 320000
D = 128
H = 128
NUM_MASK = 5000

N_PAD = 10240
NC = 2
NS = 16
NW = NC * NS
R_PT = N_PAD // NS
CH = 128
NCH = 80
E_PT = NCH * CH
E_PAD = NW * E_PT
G_PAD = 5120
G_PT = G_PAD // NW
GCH = 80

_EPAD_IDX = (np.arange(E_PAD - E, dtype=np.int32) % (N_PAD - N)) + N


def _threefry2x32_np(k1, k2, x0, x1):
    x0 = x0.astype(np.uint32).copy()
    x1 = x1.astype(np.uint32).copy()
    ks = [np.uint32(k1), np.uint32(k2), np.uint32(k1 ^ k2 ^ 0x1BD11BDA)]

    def rotl(v, r):
        return (v << r) | (v >> np.uint32(32 - r))

    rots = [(13, 15, 26, 6), (17, 29, 16, 24)]
    kidx = [(1, 2), (2, 0), (0, 1), (1, 2), (2, 0)]
    with np.errstate(over="ignore"):
        x0 = x0 + ks[0]
        x1 = x1 + ks[1]
        for i in range(5):
            for r in rots[i % 2]:
                x0 = x0 + x1
                x1 = rotl(x1, np.uint32(r))
                x1 = x0 ^ x1
            a, b = kidx[i]
            x0 = x0 + ks[a]
            x1 = x1 + ks[b] + np.uint32(i + 1)
    return x0, x1


def _permutation_np(seed, n):
    key = (np.uint32((seed >> 32) & 0xFFFFFFFF), np.uint32(seed & 0xFFFFFFFF))
    x = np.arange(n)
    num_rounds = int(np.ceil(3 * np.log(max(1, n))
                             / np.log(np.iinfo(np.uint32).max)))
    for _ in range(num_rounds):
        b1, b2 = _threefry2x32_np(key[0], key[1],
                                  np.array([0, 0]), np.array([0, 1]))
        key, sub = (b1[0], b2[0]), (b1[1], b2[1])
        h1, h2 = _threefry2x32_np(sub[0], sub[1],
                                  np.zeros(n, np.uint32),
                                  np.arange(n, dtype=np.uint32))
        x = x[np.argsort(h1 ^ h2, kind="stable")]
    return x


@functools.cache
def _mask_consts():
    perm = _permutation_np(42, N)
    mask_nodes = perm[:NUM_MASK].astype(np.int32)
    mask_vec = np.zeros((N_PAD, 1), np.float32)
    mask_vec[mask_nodes, 0] = 1.0
    gidx = np.concatenate(
        [mask_nodes, (np.arange(G_PAD - NUM_MASK, dtype=np.int32) * 83) % N]
    )
    return mask_vec, gidx

_f32 = jnp.float32


def _worker_id():
    return lax.axis_index("c") * NS + lax.axis_index("s")


@functools.cache
def _sc_kernels():
    mesh = plsc.VectorSubcoreMesh(core_axis_name="c", subcore_axis_name="s",
                                  num_cores=NC, num_subcores=NS)

    @functools.partial(
        pl.kernel,
        out_type=jax.ShapeDtypeStruct((NC, N_PAD, H), _f32),
        mesh=mesh,
        scratch_types=(
            [pltpu.VMEM((CH,), jnp.int32) for _ in range(8)]
            + [pltpu.VMEM((CH, H), _f32) for _ in range(2)]
            + [pltpu.VMEM_SHARED((N_PAD, H), _f32)]
            + [pltpu.SemaphoreType.DMA for _ in range(8)]
        ),
    )
    def scatter_kernel(hs_hbm, src_hbm, dst_hbm, zeros_hbm, out_hbm,
                       si0, si1, si2, si3, di0, di1, di2, di3,
                       rows0, rows1, acc_sh,
                       mi0, mi1, mi2, mi3, mg0, mg1, ms0, ms1):
        c = lax.axis_index("c")
        s = lax.axis_index("s")
        w = _worker_id()
        sidx = (si0, si1, si2, si3)
        didx = (di0, di1, di2, di3)
        rows = (rows0, rows1)
        semi = (mi0, mi1, mi2, mi3)
        semg = (mg0, mg1)
        sems = (ms0, ms1)

        def start_idx(k, q):
            off = w * E_PT + k * CH
            pltpu.async_copy(src_hbm.at[pl.ds(off, CH)], sidx[q], semi[q])
            pltpu.async_copy(dst_hbm.at[pl.ds(off, CH)], didx[q], semi[q])

        def wait_idx(k, q):
            off = w * E_PT + k * CH
            pltpu.make_async_copy(src_hbm.at[pl.ds(off, CH)], sidx[q],
                                  semi[q]).wait()
            pltpu.make_async_copy(dst_hbm.at[pl.ds(off, CH)], didx[q],
                                  semi[q]).wait()

        def run_chunk(k, q, b, first):
            wait_idx(k, q)
            if not first:
                pltpu.make_async_copy(rows[b], acc_sh.at[didx[q]],
                                      sems[b]).wait()
            pltpu.async_copy(hs_hbm.at[sidx[q]], rows[b], semg[b]).wait()
            pltpu.async_copy(rows[b], acc_sh.at[didx[q]], sems[b], add=True)

        pltpu.sync_copy(zeros_hbm.at[pl.ds(s * R_PT, R_PT)],
                        acc_sh.at[pl.ds(s * R_PT, R_PT)])
        plsc.subcore_barrier()

        start_idx(0, 0)
        start_idx(1, 1)
        for k in range(4):
            run_chunk(k, k % 4, k % 2, first=k < 2)
            start_idx(k + 2, (k + 2) % 4)

        def body(j, carry):
            for i in range(4):
                k = 4 * j + 4 + i
                run_chunk(k, i, i % 2, first=False)

                @pl.when(k + 2 < NCH)
                def _():
                    start_idx(k + 2, (i + 2) % 4)
            return carry

        lax.fori_loop(0, (NCH - 4) // 4, body, 0)
        for b in (0, 1):
            pltpu.make_async_copy(rows[b], acc_sh.at[didx[b]], sems[b]).wait()
        plsc.subcore_barrier()
        pltpu.sync_copy(acc_sh.at[pl.ds(s * R_PT, R_PT)],
                        out_hbm.at[c, pl.ds(s * R_PT, R_PT)])

    @functools.partial(
        pl.kernel,
        out_type=jax.ShapeDtypeStruct((NC, N_PAD, H), _f32),
        mesh=mesh,
        scratch_types=(
            [pltpu.VMEM((CH,), jnp.int32) for _ in range(4)]
            + [pltpu.VMEM((CH, H), _f32)]
            + [pltpu.VMEM_SHARED((N_PAD, H), _f32)]
            + [pltpu.SemaphoreType.DMA for _ in range(6)]
        ),
    )
    def deg_kernel(dst_hbm, ones_hbm, zeros_hbm, out_hbm,
                   di0, di1, di2, di3, ones_v, acc_sh,
                   mi0, mi1, mi2, mi3, ms0, ms1):
        c = lax.axis_index("c")
        s = lax.axis_index("s")
        w = _worker_id()
        didx = (di0, di1, di2, di3)
        semi = (mi0, mi1, mi2, mi3)
        sems = (ms0, ms1)

        def start_idx(k, q):
            off = w * E_PT + k * CH
            pltpu.async_copy(dst_hbm.at[pl.ds(off, CH)], didx[q], semi[q])

        def wait_idx(k, q):
            off = w * E_PT + k * CH
            pltpu.make_async_copy(dst_hbm.at[pl.ds(off, CH)], didx[q],
                                  semi[q]).wait()

        def run_chunk(k, q, b, first):
            wait_idx(k, q)
            if not first:
                pltpu.make_async_copy(ones_v, acc_sh.at[didx[q]],
                                      sems[b]).wait()
            pltpu.async_copy(ones_v, acc_sh.at[didx[q]], sems[b], add=True)

        pltpu.sync_copy(zeros_hbm.at[pl.ds(s * R_PT, R_PT)],
                        acc_sh.at[pl.ds(s * R_PT, R_PT)])
        pltpu.sync_copy(ones_hbm, ones_v)
        plsc.subcore_barrier()

        start_idx(0, 0)
        start_idx(1, 1)
        for k in range(4):
            run_chunk(k, k % 4, k % 2, first=k < 2)
            start_idx(k + 2, (k + 2) % 4)

        def body(j, carry):
            for i in range(4):
                k = 4 * j + 4 + i
                run_chunk(k, i, i % 2, first=False)

                @pl.when(k + 2 < NCH)
                def _():
                    start_idx(k + 2, (i + 2) % 4)
            return carry

        lax.fori_loop(0, (NCH - 4) // 4, body, 0)
        for b in (0, 1):
            pltpu.make_async_copy(ones_v, acc_sh.at[didx[b]], sems[b]).wait()
        plsc.subcore_barrier()
        pltpu.sync_copy(acc_sh.at[pl.ds(s * R_PT, R_PT)],
                        out_hbm.at[c, pl.ds(s * R_PT, R_PT)])

    @functools.partial(
        pl.kernel,
        out_type=(jax.ShapeDtypeStruct((G_PAD, D), _f32),
                  jax.ShapeDtypeStruct((G_PAD, D), _f32)),
        mesh=mesh,
        scratch_types=[
            pltpu.VMEM((GCH,), jnp.int32),
            pltpu.VMEM((GCH, D), _f32),
            pltpu.SemaphoreType.DMA,
        ],
    )
    def gather_kernel(x_hbm, r_hbm, gidx_hbm, out0_hbm, out1_hbm, idx_v, rows_v, sem):
        w = _worker_id()
        for part in range(G_PT // GCH):
            base = w * G_PT + part * GCH
            pltpu.sync_copy(gidx_hbm.at[pl.ds(base, GCH)], idx_v)
            pltpu.async_copy(x_hbm.at[idx_v], rows_v, sem).wait()
            pltpu.sync_copy(rows_v, out0_hbm.at[pl.ds(base, GCH)])
            pltpu.async_copy(r_hbm.at[idx_v], rows_v, sem).wait()
            pltpu.sync_copy(rows_v, out1_hbm.at[pl.ds(base, GCH)])

    return deg_kernel, scatter_kernel, gather_kernel


R_TC = 256
G_TC = N_PAD // R_TC


def _row_spec(width):
    return pl.BlockSpec((R_TC, width), lambda i: (i, 0))


def _full_spec(shape):
    nd = len(shape)
    return pl.BlockSpec(shape, lambda i: (0,) * nd)


def _pre_body(x_ref, m_ref, mt_ref, w_ref, dp_ref, hs_ref, dinv_ref):
    i = pl.program_id(0)
    dp = dp_ref[...]
    deg = dp[0, :, 0:1] + dp[1, :, 0:1] + 1.0
    rows = lax.broadcasted_iota(jnp.int32, (R_TC, 1), 0) + i * R_TC
    dinv = lax.rsqrt(deg) * (rows < N).astype(_f32)
    m = m_ref[...]
    z0 = x_ref[...] * (1.0 - m) + mt_ref[...] * m
    h = jnp.dot(z0, w_ref[...], preferred_element_type=_f32)
    hs_ref[...] = h * dinv
    dinv_ref[...] = dinv


def _tc_pre(x_pad, mvec, mt, W0, deg_col):
    return pl.pallas_call(
        _pre_body,
        grid=(G_TC,),
        in_specs=[
            _row_spec(D),
            _row_spec(1),
            _full_spec((1, D)),
            _full_spec((D, H)),
            pl.BlockSpec((NC, R_TC, H), lambda i: (0, i, 0)),
        ],
        out_specs=(_row_spec(H), _row_spec(1)),
        out_shape=(jax.ShapeDtypeStruct((N_PAD, H), _f32),
                   jax.ShapeDtypeStruct((N_PAD, 1), _f32)),
    )(x_pad, mvec, mt, W0, deg_col)


def _mid_body(p_ref, hs_ref, dinv_ref, b_ref, w_ref, out_ref):
    p = p_ref[...]
    acc = p[0] + p[1] + hs_ref[...]
    dinv = dinv_ref[...]
    z = jnp.maximum(dinv * acc + b_ref[...], 0.0)
    out_ref[...] = jnp.dot(z, w_ref[...], preferred_element_type=_f32) * dinv


def _tc_mid(p, hs_prev, dinv, b_prev, W_next):
    return pl.pallas_call(
        _mid_body,
        grid=(G_TC,),
        in_specs=[
            pl.BlockSpec((NC, R_TC, H), lambda i: (0, i, 0)),
            _row_spec(H),
            _row_spec(1),
            _full_spec((1, H)),
            _full_spec((H, H)),
        ],
        out_specs=_row_spec(H),
        out_shape=jax.ShapeDtypeStruct((N_PAD, H), _f32),
    )(p, hs_prev, dinv, b_prev, W_next)


def _post_body(p_ref, hs_ref, dinv_ref, b_ref, w0_ref, c0_ref, w1_ref, c1_ref,
               w2_ref, c2_ref, out_ref):
    p = p_ref[...]
    acc = p[0] + p[1] + hs_ref[...]
    z = jnp.maximum(dinv_ref[...] * acc + b_ref[...], 0.0)
    r = jnp.maximum(jnp.dot(z, w0_ref[...], preferred_element_type=_f32)
                    + c0_ref[...], 0.0)
    r = jnp.maximum(jnp.dot(r, w1_ref[...], preferred_element_type=_f32)
                    + c1_ref[...], 0.0)
    out_ref[...] = (jnp.dot(r, w2_ref[...], preferred_element_type=_f32)
                    + c2_ref[...])


def _tc_post(p, hs3, dinv, b2, Dw0, Db0, Dw1, Db1, Dw2, Db2):
    return pl.pallas_call(
        _post_body,
        grid=(G_TC,),
        in_specs=[
            pl.BlockSpec((NC, R_TC, H), lambda i: (0, i, 0)),
            _row_spec(H),
            _row_spec(1),
            _full_spec((1, H)),
            _full_spec((H, H)),
            _full_spec((1, H)),
            _full_spec((H, H)),
            _full_spec((1, H)),
            _full_spec((H, D)),
            _full_spec((1, D)),
        ],
        out_specs=_row_spec(D),
        out_shape=jax.ShapeDtypeStruct((N_PAD, D), _f32),
    )(p, hs3, dinv, b2, Dw0, Db0, Dw1, Db1, Dw2, Db2)


def kernel(x, edge_index, W0, b0, W1, b1, W2, b2,
           Dw0, Db0, Dw1, Db1, Dw2, Db2, mask_token):
    x_pad = jnp.concatenate([x, jnp.zeros((N_PAD - N, D), _f32)], axis=0)
    src = jnp.concatenate([edge_index[0], jnp.asarray(_EPAD_IDX)])
    dst = jnp.concatenate([edge_index[1], jnp.asarray(_EPAD_IDX)])
    mask_vec, gidx_np = _mask_consts()
    mvec = jnp.asarray(mask_vec)
    gidx = jnp.asarray(gidx_np)
    ones_ch = jnp.ones((CH, H), _f32)
    zerosH = jnp.zeros((N_PAD, H), _f32)

    _deg_kernel, _scatter_kernel, _gather_kernel = _sc_kernels()
    degp = _deg_kernel(dst, ones_ch, zerosH)
    hs1, dinv = _tc_pre(x_pad, mvec, mask_token, W0, degp)
    p1 = _scatter_kernel(hs1, src, dst, zerosH)
    hs2 = _tc_mid(p1, hs1, dinv, b0.reshape(1, H), W1)
    p2 = _scatter_kernel(hs2, src, dst, zerosH)
    hs3 = _tc_mid(p2, hs2, dinv, b1.reshape(1, H), W2)
    p3 = _scatter_kernel(hs3, src, dst, zerosH)
    r = _tc_post(p3, hs3, dinv, b2.reshape(1, H),
                 Dw0, Db0.reshape(1, H), Dw1, Db1.reshape(1, H),
                 Dw2, Db2.reshape(1, D))
    out0, out1 = _gather_kernel(x_pad, r, gidx)
    return (out0[:NUM_MASK], out1[:NUM_MASK])
